# Initial kernel scaffold; baseline (speedup 1.0000x reference)
import functools

import jax
import jax.numpy as jnp
from jax import lax
from jax.experimental import pallas as pl
from jax.experimental.pallas import tpu as pltpu
from jax.experimental.pallas import tpu_sc as plsc

L_WALK = 80
N_IDX = 241
N_PAD = 256
DIM = 128
NSUB = 16
ROWS_PER_W = N_PAD // NSUB
NCHUNK = DIM // 16
LN2 = 0.6931471805599453


def _body(rw_hbm, x_hbm, out_hbm,
          idx_v, sidx_v, rows_v, srow_v, dtmp_v,
          dots_v, wbuf_v, res_v, dots_sh, sem):
    c = lax.axis_index("c")
    s = lax.axis_index("s")
    lane = lax.broadcasted_iota(jnp.int32, (16,), 0)

    @pl.when(c == 0)
    def _gather_and_dot():
        base = s * ROWS_PER_W
        pltpu.sync_copy(rw_hbm.at[pl.ds(base, ROWS_PER_W)], idx_v)
        pltpu.sync_copy(rw_hbm.at[pl.ds(0, 8)], sidx_v)
        pltpu.async_copy(x_hbm.at[idx_v], rows_v, sem).wait()
        pltpu.async_copy(x_hbm.at[sidx_v.at[pl.ds(0, 1)]], srow_v, sem).wait()

        xs = [srow_v[0, pl.ds(k * 16, 16)] for k in range(NCHUNK)]
        dvec = jnp.zeros((16,), jnp.float32)
        for r in range(ROWS_PER_W):
            acc = rows_v[r, pl.ds(0, 16)] * xs[0]
            for k in range(1, NCHUNK):
                acc = acc + rows_v[r, pl.ds(k * 16, 16)] * xs[k]
            dr = jnp.sum(acc)
            dvec = jnp.where(lane == r, dr, dvec)
        dtmp_v[...] = dvec
        pltpu.sync_copy(dtmp_v, dots_sh.at[s])

    plsc.subcore_barrier()

    @pl.when((c == 0) & (s == 0))
    def _combine():
        pltpu.sync_copy(dots_sh, dots_v)
        for cc in range(6):
            wbuf_v[pl.ds(cc * 16, 16)] = jnp.full((16,), -1, jnp.int32)
        pltpu.sync_copy(rw_hbm.at[pl.ds(0, 96)], wbuf_v.at[pl.ds(96, 96)])

        dup = [jnp.zeros((16,), jnp.bool_) for _ in range(6)]
        cur = [wbuf_v[pl.ds(96 + cc * 16, 16)] for cc in range(6)]
        for sh in range(1, L_WALK + 1):
            for cc in range(6):
                if cc * 16 + 15 < sh:
                    continue
                if cc * 16 > L_WALK:
                    continue
                prev = wbuf_v[pl.ds(96 + cc * 16 - sh, 16)]
                dup[cc] = dup[cc] | (prev == cur[cc])

        total = jnp.zeros((16,), jnp.float32)
        nacc = jnp.zeros((16,), jnp.float32)
        for cc in range(16):
            dch = dots_v[cc]
            gi = lane + cc * 16
            if cc < 6:
                keep = jnp.logical_not(dup[cc]) | (gi > L_WALK)
                nacc = nacc + jnp.where((gi >= 1) & (gi <= L_WALK), dch, 0.0)
            else:
                keep = gi < N_IDX
            total = total + jnp.where(keep, jnp.exp(dch), 0.0)
        denom = jnp.sum(total)
        numer = jnp.sum(nacc)

        dv = jnp.full((16,), denom)
        bits = plsc.bitcast(dv, jnp.int32)
        e = (bits >> 23) - 127
        m = plsc.bitcast((bits & 0x007FFFFF) | 0x3F800000, jnp.float32)
        adj = m > 1.4142135623730951
        mm = jnp.where(adj, m * 0.5, m)
        ef = (e + jnp.where(adj, 1, 0)).astype(jnp.float32)
        t = (mm - 1.0) / (mm + 1.0)
        t2 = t * t
        ln_m = 2.0 * t * (1.0 + t2 * (1.0 / 3.0 + t2 * (0.2 + t2 * (1.0 / 7.0))))
        ln_x = ef * LN2 + ln_m
        res_v[...] = float(L_WALK) * ln_x - numer
        pltpu.sync_copy(res_v, out_hbm)


def _n2v(rw_pad, x):
    mesh = plsc.VectorSubcoreMesh(core_axis_name="c", subcore_axis_name="s")
    f = pl.kernel(
        _body,
        out_type=jax.ShapeDtypeStruct((16,), jnp.float32),
        mesh=mesh,
        scratch_types=[
            pltpu.VMEM((ROWS_PER_W,), jnp.int32),
            pltpu.VMEM((8,), jnp.int32),
            pltpu.VMEM((ROWS_PER_W, DIM), jnp.float32),
            pltpu.VMEM((1, DIM), jnp.float32),
            pltpu.VMEM((16,), jnp.float32),
            pltpu.VMEM((NSUB, 16), jnp.float32),
            pltpu.VMEM((192,), jnp.int32),
            pltpu.VMEM((16,), jnp.float32),
            pltpu.VMEM_SHARED((NSUB, 16), jnp.float32),
            pltpu.SemaphoreType.DMA,
        ],
    )
    return f(rw_pad, x)


def kernel(rw_vec, X):
    rw_pad = jnp.concatenate(
        [rw_vec.astype(jnp.int32), jnp.zeros((N_PAD - N_IDX,), jnp.int32)])
    return _n2v(rw_pad, X)[0]


# speedup vs baseline: 1.9016x; 1.9016x over previous
import functools

import jax
import jax.numpy as jnp
from jax import lax
from jax.experimental import pallas as pl
from jax.experimental.pallas import tpu as pltpu
from jax.experimental.pallas import tpu_sc as plsc

L_WALK = 80
N_IDX = 241
N_PAD = 256
DIM = 128
NSUB = 16
ROWS_PER_W = N_PAD // NSUB
NCHUNK = DIM // 16
LN2 = 0.6931471805599453


def _body(rw_hbm, x_hbm, out_hbm,
          idx_v, sidx_v, rows_v, srow_v, dtmp_v,
          dots_v, wbuf_v, res_v, dots_sh, sem):
    c = lax.axis_index("c")
    s = lax.axis_index("s")
    lane = lax.broadcasted_iota(jnp.int32, (16,), 0)

    @pl.when(c == 0)
    def _gather_and_dot():
        base = s * ROWS_PER_W
        pltpu.sync_copy(rw_hbm.at[pl.ds(base, ROWS_PER_W)], idx_v)
        pltpu.sync_copy(rw_hbm.at[pl.ds(0, 8)], sidx_v)
        pltpu.async_copy(x_hbm.at[idx_v], rows_v, sem).wait()
        pltpu.async_copy(x_hbm.at[sidx_v.at[pl.ds(0, 1)]], srow_v, sem).wait()

        xs = [srow_v[0, pl.ds(k * 16, 16)] for k in range(NCHUNK)]
        dvec = jnp.zeros((16,), jnp.float32)
        for r in range(ROWS_PER_W):
            acc = rows_v[r, pl.ds(0, 16)] * xs[0]
            for k in range(1, NCHUNK):
                acc = acc + rows_v[r, pl.ds(k * 16, 16)] * xs[k]
            dvec = jnp.where(lane == r, jnp.sum(acc), dvec)
        dtmp_v[...] = dvec
        pltpu.sync_copy(dtmp_v, dots_sh.at[s])

    plsc.subcore_barrier()

    @pl.when((c == 0) & (s == 0))
    def _combine():
        pltpu.sync_copy(dots_sh, dots_v)
        for cc in range(6):
            wbuf_v[pl.ds(cc * 16, 16)] = jnp.full((16,), -1, jnp.int32)
        pltpu.sync_copy(rw_hbm.at[pl.ds(0, 96)], wbuf_v.at[pl.ds(96, 96)])

        dup = [jnp.zeros((16,), jnp.bool_) for _ in range(6)]
        cur = [wbuf_v[pl.ds(96 + cc * 16, 16)] for cc in range(6)]
        for sh in range(1, L_WALK + 1):
            for cc in range(6):
                if cc * 16 + 15 < sh:
                    continue
                if cc * 16 > L_WALK:
                    continue
                prev = wbuf_v[pl.ds(96 + cc * 16 - sh, 16)]
                dup[cc] = dup[cc] | (prev == cur[cc])

        total = jnp.zeros((16,), jnp.float32)
        nacc = jnp.zeros((16,), jnp.float32)
        for cc in range(16):
            dch = dots_v[cc]
            gi = lane + cc * 16
            if cc < 6:
                keep = jnp.logical_not(dup[cc]) | (gi > L_WALK)
                nacc = nacc + jnp.where((gi >= 1) & (gi <= L_WALK), dch, 0.0)
            else:
                keep = gi < N_IDX
            total = total + jnp.where(keep, jnp.exp(dch), 0.0)
        denom = jnp.full((16,), jnp.sum(total))
        numer = jnp.sum(nacc)

        bits = plsc.bitcast(denom, jnp.int32)
        e = (bits >> 23) - 127
        m = plsc.bitcast((bits & 0x007FFFFF) | 0x3F800000, jnp.float32)
        adj = m > 1.4142135623730951
        mm = jnp.where(adj, m * 0.5, m)
        ef = (e + jnp.where(adj, 1, 0)).astype(jnp.float32)
        t = (mm - 1.0) / (mm + 1.0)
        t2 = t * t
        ln_m = 2.0 * t * (1.0 + t2 * (1.0 / 3.0 + t2 * (0.2 + t2 * (1.0 / 7.0))))
        ln_x = ef * LN2 + ln_m
        res_v[...] = float(L_WALK) * ln_x - numer
        pltpu.sync_copy(res_v, out_hbm)


def _n2v(rw_pad, x):
    mesh = plsc.VectorSubcoreMesh(core_axis_name="c", subcore_axis_name="s")
    f = pl.kernel(
        _body,
        out_type=jax.ShapeDtypeStruct((16,), jnp.float32),
        mesh=mesh,
        scratch_types=[
            pltpu.VMEM((ROWS_PER_W,), jnp.int32),
            pltpu.VMEM((8,), jnp.int32),
            pltpu.VMEM((ROWS_PER_W, DIM), jnp.float32),
            pltpu.VMEM((1, DIM), jnp.float32),
            pltpu.VMEM((16,), jnp.float32),
            pltpu.VMEM((NSUB, 16), jnp.float32),
            pltpu.VMEM((192,), jnp.int32),
            pltpu.VMEM((16,), jnp.float32),
            pltpu.VMEM_SHARED((NSUB, 16), jnp.float32),
            pltpu.SemaphoreType.DMA,
        ],
        compiler_params=pltpu.CompilerParams(needs_layout_passes=False),
    )
    return f(rw_pad, x)


def kernel(rw_vec, X):
    rw_pad = jnp.concatenate(
        [rw_vec.astype(jnp.int32), jnp.zeros((N_PAD - N_IDX,), jnp.int32)])
    return _n2v(rw_pad, X)[0]


# speedup vs baseline: 2.0386x; 1.0721x over previous
import jax
import jax.numpy as jnp
from jax import lax
from jax.experimental import pallas as pl
from jax.experimental.pallas import tpu as pltpu
from jax.experimental.pallas import tpu_sc as plsc

L_WALK = 80
N_IDX = 241
DIM = 128
NSUB = 16
NCHUNK = DIM // 16
LN2 = 0.6931471805599453


def _body(rw_hbm, x_hbm, out_hbm,
          idx_v, rows_v, wbuf_v, part_v, part_all_v, res_v,
          part_sh, sem):
    c = lax.axis_index("c")
    s = lax.axis_index("s")
    lane = lax.broadcasted_iota(jnp.int32, (16,), 0)

    @pl.when(c == 0)
    def _work():
        @pl.when(s < 15)
        def _():
            d1 = pltpu.async_copy(rw_hbm.at[pl.ds(0, 8)],
                                  idx_v.at[pl.ds(0, 8)], sem)
            d2 = pltpu.async_copy(rw_hbm.at[pl.ds(s * 16, 16)],
                                  idx_v.at[pl.ds(8, 16)], sem)
            d1.wait()
            d2.wait()

        @pl.when(s == 15)
        def _():
            idx_v[pl.ds(8, 16)] = jnp.zeros((16,), jnp.int32)
            d1 = pltpu.async_copy(rw_hbm.at[pl.ds(0, 8)],
                                  idx_v.at[pl.ds(0, 8)], sem)
            d2 = pltpu.async_copy(rw_hbm.at[pl.ds(240, 1)],
                                  idx_v.at[pl.ds(8, 1)], sem)
            d1.wait()
            d2.wait()

        g = pltpu.async_copy(x_hbm.at[idx_v], rows_v, sem)
        for cc in range(6):
            wbuf_v[pl.ds(cc * 16, 16)] = jnp.full((16,), -1, jnp.int32)
        wb = pltpu.async_copy(rw_hbm.at[pl.ds(0, 96)],
                              wbuf_v.at[pl.ds(96, 96)], sem)
        g.wait()

        xs = [rows_v[0, pl.ds(k * 16, 16)] for k in range(NCHUNK)]
        dvec = jnp.zeros((16,), jnp.float32)
        for r in range(16):
            acc = rows_v[8 + r, pl.ds(0, 16)] * xs[0]
            for k in range(1, NCHUNK):
                acc = acc + rows_v[8 + r, pl.ds(k * 16, 16)] * xs[k]
            dvec = jnp.where(lane == r, jnp.sum(acc), dvec)

        wb.wait()
        sbase = jnp.minimum(s, 5) * 16
        cur = wbuf_v[pl.ds(96 + sbase, 16)]
        dup = jnp.zeros((16,), jnp.bool_)
        for sh in range(1, L_WALK + 1):
            prev = wbuf_v[pl.ds(96 + sbase - sh, 16)]
            dup = dup | (prev == cur)

        gi = lane + s * 16
        keep = jnp.where(s < 6,
                         jnp.logical_not(dup) | (gi > L_WALK),
                         gi < N_IDX)
        part_v[0] = jnp.where(keep, jnp.exp(dvec), 0.0)
        part_v[1] = jnp.where((gi >= 1) & (gi <= L_WALK), dvec, 0.0)
        pltpu.sync_copy(part_v, part_sh.at[s])

    plsc.subcore_barrier()

    @pl.when((c == 0) & (s == 0))
    def _combine():
        pltpu.sync_copy(part_sh, part_all_v)
        total = part_all_v[0, 0]
        for cc in range(1, 16):
            total = total + part_all_v[cc, 0]
        nacc = part_all_v[0, 1]
        for cc in range(1, 6):
            nacc = nacc + part_all_v[cc, 1]
        denom = jnp.full((16,), jnp.sum(total))
        numer = jnp.sum(nacc)

        bits = plsc.bitcast(denom, jnp.int32)
        e = (bits >> 23) - 127
        m = plsc.bitcast((bits & 0x007FFFFF) | 0x3F800000, jnp.float32)
        adj = m > 1.4142135623730951
        mm = jnp.where(adj, m * 0.5, m)
        ef = (e + jnp.where(adj, 1, 0)).astype(jnp.float32)
        t = (mm - 1.0) / (mm + 1.0)
        t2 = t * t
        ln_m = 2.0 * t * (1.0 + t2 * (1.0 / 3.0 + t2 * (0.2 + t2 * (1.0 / 7.0))))
        ln_x = ef * LN2 + ln_m
        res_v[...] = float(L_WALK) * ln_x - numer
        pltpu.sync_copy(res_v, out_hbm)


def _n2v(rw_vec, x):
    mesh = plsc.VectorSubcoreMesh(core_axis_name="c", subcore_axis_name="s")
    f = pl.kernel(
        _body,
        out_type=jax.ShapeDtypeStruct((16,), jnp.float32),
        mesh=mesh,
        scratch_types=[
            pltpu.VMEM((24,), jnp.int32),
            pltpu.VMEM((24, DIM), jnp.float32),
            pltpu.VMEM((192,), jnp.int32),
            pltpu.VMEM((2, 16), jnp.float32),
            pltpu.VMEM((NSUB, 2, 16), jnp.float32),
            pltpu.VMEM((16,), jnp.float32),
            pltpu.VMEM_SHARED((NSUB, 2, 16), jnp.float32),
            pltpu.SemaphoreType.DMA,
        ],
        compiler_params=pltpu.CompilerParams(needs_layout_passes=False),
    )
    return f(rw_vec, x)


def kernel(rw_vec, X):
    return _n2v(rw_vec.astype(jnp.int32), X)[0]
